# baseline (speedup 1.0000x reference)
import jax
import jax.numpy as jnp
from jax import lax
from jax.experimental import pallas as pl
from jax.experimental.pallas import tpu as pltpu
from jax.experimental.pallas import tpu_sc as plsc

N_NODES = 10000
N_EDGES = 320000
IN_CH = 128
HIDDEN = 64
OUT_CH = 32
NUM_GRAPHS = 128

NC = 2
NS = 16
NW = NC * NS
EPW = N_EDGES // NW
K = 128
EPWP = 10240
NCHUNK = EPWP // K
NBUF = 4
N_EPAD = NW * EPWP - N_EDGES
NPAD = 10240
RPT = NPAD // NS
HRPT = RPT // 2
WROW = HIDDEN

_f32 = jnp.float32
_i32 = jnp.int32

_sc_mesh = plsc.VectorSubcoreMesh(core_axis_name="c", subcore_axis_name="s")
_sc_params = pltpu.CompilerParams(needs_layout_passes=False,
                                 use_tc_tiling_on_sc=False)


def _deg_body(dst_hbm, out_hbm, dstv, hist):
    c = lax.axis_index("c")
    s = lax.axis_index("s")
    w = c * NS + s

    zero = jnp.zeros((16,), _f32)

    def z(i, carry):
        hist[pl.ds(i * 16, 16)] = zero
        return carry

    lax.fori_loop(0, NPAD // 16, z, 0)
    pltpu.sync_copy(dst_hbm.at[w], dstv)

    ones = jnp.full((16,), 1.0, _f32)

    def body(e, carry):
        idx = dstv[pl.ds(e * 16, 16)]
        plsc.addupdate_scatter(hist, [idx], ones)
        return carry

    lax.fori_loop(0, EPWP // 16, body, 0)
    pltpu.sync_copy(hist, out_hbm.at[w])


_deg_call = pl.kernel(
    _deg_body,
    out_type=jax.ShapeDtypeStruct((NW, NPAD), _f32),
    mesh=_sc_mesh,
    compiler_params=_sc_params,
    scratch_types=[
        pltpu.VMEM((EPWP,), _i32),
        pltpu.VMEM((NPAD,), _f32),
    ],
)


def _agg_body(ls_hbm, src_hbm, dst_hbm, zrows_hbm, out_hbm,
              idxv, dstv, rows0, rows1, obuf, acc, gs0, gs1):
    c = lax.axis_index("c")
    s = lax.axis_index("s")
    w = c * NS + s

    pltpu.sync_copy(zrows_hbm, obuf)
    for m in range(2):
        pltpu.sync_copy(obuf, acc.at[pl.ds(s * RPT + m * HRPT, HRPT)])
    pltpu.sync_copy(src_hbm.at[w], idxv)
    pltpu.sync_copy(dst_hbm.at[w], dstv)
    plsc.subcore_barrier()

    def pair(i, carry):
        j0 = 2 * i
        d0 = pltpu.async_copy(ls_hbm.at[idxv.at[j0]], rows0, gs0)
        d1 = pltpu.async_copy(ls_hbm.at[idxv.at[j0 + 1]], rows1, gs1)
        d0.wait()
        pltpu.sync_copy(rows0, acc.at[dstv.at[j0]], add=True)
        d1.wait()
        pltpu.sync_copy(rows1, acc.at[dstv.at[j0 + 1]], add=True)
        return carry

    lax.fori_loop(0, NCHUNK // 2, pair, 0)
    plsc.subcore_barrier()

    for m in range(2):
        pltpu.sync_copy(acc.at[pl.ds(s * RPT + m * HRPT, HRPT)], obuf)
        pltpu.sync_copy(obuf, out_hbm.at[c, pl.ds(s * RPT + m * HRPT, HRPT)])


_agg_call = pl.kernel(
    _agg_body,
    out_type=jax.ShapeDtypeStruct((NC, NPAD, WROW), _f32),
    mesh=_sc_mesh,
    compiler_params=_sc_params,
    scratch_types=[
        pltpu.VMEM((NCHUNK, K), _i32),
        pltpu.VMEM((NCHUNK, K), _i32),
        pltpu.VMEM((K, WROW), _f32),
        pltpu.VMEM((K, WROW), _f32),
        pltpu.VMEM((HRPT, WROW), _f32),
        pltpu.VMEM_SHARED((NPAD, WROW), _f32),
        pltpu.SemaphoreType.DMA,
        pltpu.SemaphoreType.DMA,
    ],
)


def _dot(a, b):
    return lax.dot_general(a, b, (((1,), (0,)), ((), ())),
                           precision=lax.Precision.HIGHEST,
                           preferred_element_type=_f32)


def _dot_t(a, b):
    return lax.dot_general(a, b, (((0,), (0,)), ((), ())),
                           precision=lax.Precision.HIGHEST,
                           preferred_element_type=_f32)


def _tc1_body(x_ref, w1_ref, degp_ref, ones_ref, ls_ref, dinv_ref):
    deg = _dot_t(degp_ref[...], ones_ref[...])[:N_NODES] + 1.0
    dinv = lax.rsqrt(deg)
    lin = _dot(x_ref[...], w1_ref[...])
    ls_ref[...] = lin * dinv
    dinv_ref[...] = dinv


_tc1_call = pl.pallas_call(
    _tc1_body,
    out_shape=[jax.ShapeDtypeStruct((N_NODES, WROW), _f32),
               jax.ShapeDtypeStruct((N_NODES, 1), _f32)],
)


def _tc2_body(a0_ref, a1_ref, ls1_ref, dinv_ref, b1_ref, w2_ref, ls2_ref):
    dinv = dinv_ref[...]
    a = a0_ref[:N_NODES, :HIDDEN] + a1_ref[:N_NODES, :HIDDEN]
    h = dinv * (a + ls1_ref[:N_NODES, :HIDDEN]) + b1_ref[...]
    h = jnp.maximum(h, 0.0)
    ls2_ref[...] = _dot(h, w2_ref[...]) * dinv


_tc2_call = pl.pallas_call(
    _tc2_body,
    out_shape=jax.ShapeDtypeStruct((N_NODES, WROW), _f32),
)


def _tc3_body(a0_ref, a1_ref, ls2_ref, dinv_ref, b2_ref, batch_ref,
              wfc_ref, bfc_ref, out_ref):
    dinv = dinv_ref[...]
    a = a0_ref[:N_NODES, :HIDDEN] + a1_ref[:N_NODES, :HIDDEN]
    h = dinv * (a + ls2_ref[:N_NODES, :HIDDEN]) + b2_ref[...]
    h = jnp.maximum(h, 0.0)
    gid = lax.broadcasted_iota(_i32, (NUM_GRAPHS, N_NODES), 0)
    mask = (batch_ref[...] == gid).astype(_f32)
    sums = _dot(mask, h)
    cnts = jnp.sum(mask, axis=1, keepdims=True)
    pooled = sums / jnp.maximum(cnts, 1.0)
    out_ref[...] = jnp.maximum(_dot(pooled, wfc_ref[...]) + bfc_ref[...], 0.0)


_tc3_call = pl.pallas_call(
    _tc3_body,
    out_shape=jax.ShapeDtypeStruct((NUM_GRAPHS, OUT_CH), _f32),
)


def kernel(x, edge_index, batch, W1, b1, W2, b2, Wfc, bfc):
    src = edge_index[0]
    dst = edge_index[1]
    src_p = jnp.concatenate([src, jnp.zeros((N_EPAD,), _i32)])
    dst_p = jnp.concatenate([dst, jnp.full((N_EPAD,), N_NODES, _i32)])
    src3 = src_p.reshape(NW, NCHUNK, K)
    dst3 = dst_p.reshape(NW, NCHUNK, K)

    dst2 = dst_p.reshape(NW, EPWP)
    zrows = jnp.zeros((HRPT, WROW), _f32)
    ones_nw = jnp.ones((NW, 1), _f32)

    degp = _deg_call(dst2)
    ls1, dinv = _tc1_call(x, W1, degp, ones_nw)
    acc1 = _agg_call(ls1, src3, dst3, zrows)
    ls2 = _tc2_call(acc1[0], acc1[1], ls1, dinv, b1.reshape(1, -1), W2)
    acc2 = _agg_call(ls2, src3, dst3, zrows)
    out = _tc3_call(acc2[0], acc2[1], ls2, dinv, b2.reshape(1, -1),
                    batch.reshape(1, -1), Wfc, bfc.reshape(1, -1))
    return out


# speedup vs baseline: 1.0119x; 1.0119x over previous
import jax
import jax.numpy as jnp
from jax import lax
from jax.experimental import pallas as pl
from jax.experimental.pallas import tpu as pltpu
from jax.experimental.pallas import tpu_sc as plsc

N_NODES = 10000
N_EDGES = 320000
IN_CH = 128
HIDDEN = 64
OUT_CH = 32
NUM_GRAPHS = 128

NC = 2
NS = 16
NW = NC * NS
EPW = N_EDGES // NW
K = 128
EPWP = 10240
NCHUNK = EPWP // K
NBUF = 4
N_EPAD = NW * EPWP - N_EDGES
NPAD = 10240
RPT = NPAD // NS
HRPT = RPT // 2
WROW = HIDDEN

_f32 = jnp.float32
_i32 = jnp.int32

_sc_mesh = plsc.VectorSubcoreMesh(core_axis_name="c", subcore_axis_name="s")
_sc_params = pltpu.CompilerParams(needs_layout_passes=False,
                                 use_tc_tiling_on_sc=False)


def _deg_body(dst_hbm, out_hbm, dstv, hist):
    c = lax.axis_index("c")
    s = lax.axis_index("s")
    w = c * NS + s

    zero = jnp.zeros((16,), _f32)

    def z(i, carry):
        hist[pl.ds(i * 16, 16)] = zero
        return carry

    lax.fori_loop(0, NPAD // 16, z, 0)
    pltpu.sync_copy(dst_hbm.at[w], dstv)

    ones = jnp.full((16,), 1.0, _f32)

    def body(e, carry):
        idx = dstv[pl.ds(e * 16, 16)]
        plsc.addupdate_scatter(hist, [idx], ones)
        return carry

    lax.fori_loop(0, EPWP // 16, body, 0)
    pltpu.sync_copy(hist, out_hbm.at[w])


_deg_call = pl.kernel(
    _deg_body,
    out_type=jax.ShapeDtypeStruct((NW, NPAD), _f32),
    mesh=_sc_mesh,
    compiler_params=_sc_params,
    scratch_types=[
        pltpu.VMEM((EPWP,), _i32),
        pltpu.VMEM((NPAD,), _f32),
    ],
)


def _agg_body(ls_hbm, src_hbm, dst_hbm, zrows_hbm, out_hbm,
              idxv, dstv, rows0, rows1, obuf, acc, gs0, gs1):
    c = lax.axis_index("c")
    s = lax.axis_index("s")
    w = c * NS + s

    pltpu.sync_copy(zrows_hbm, obuf)
    for m in range(2):
        pltpu.sync_copy(obuf, acc.at[pl.ds(s * RPT + m * HRPT, HRPT)])
    pltpu.sync_copy(src_hbm.at[w], idxv)
    pltpu.sync_copy(dst_hbm.at[w], dstv)
    plsc.subcore_barrier()

    def pair(i, carry):
        j0 = 2 * i
        d0 = pltpu.async_copy(ls_hbm.at[idxv.at[j0]], rows0, gs0)
        d1 = pltpu.async_copy(ls_hbm.at[idxv.at[j0 + 1]], rows1, gs1)
        d0.wait()
        pltpu.sync_copy(rows0, acc.at[dstv.at[j0]], add=True)
        d1.wait()
        pltpu.sync_copy(rows1, acc.at[dstv.at[j0 + 1]], add=True)
        return carry

    lax.fori_loop(0, NCHUNK // 2, pair, 0)
    plsc.subcore_barrier()

    for m in range(2):
        pltpu.sync_copy(acc.at[pl.ds(s * RPT + m * HRPT, HRPT)], obuf)
        pltpu.sync_copy(obuf, out_hbm.at[c, pl.ds(s * RPT + m * HRPT, HRPT)])


_agg_call = pl.kernel(
    _agg_body,
    out_type=jax.ShapeDtypeStruct((NC, NPAD, WROW), _f32),
    mesh=_sc_mesh,
    compiler_params=_sc_params,
    scratch_types=[
        pltpu.VMEM((NCHUNK, K), _i32),
        pltpu.VMEM((NCHUNK, K), _i32),
        pltpu.VMEM((K, WROW), _f32),
        pltpu.VMEM((K, WROW), _f32),
        pltpu.VMEM((HRPT, WROW), _f32),
        pltpu.VMEM_SHARED((NPAD, WROW), _f32),
        pltpu.SemaphoreType.DMA,
        pltpu.SemaphoreType.DMA,
    ],
)


def _dot(a, b):
    return lax.dot_general(a, b, (((1,), (0,)), ((), ())),
                           precision=lax.Precision.HIGHEST,
                           preferred_element_type=_f32)


def _dot_t(a, b):
    return lax.dot_general(a, b, (((0,), (0,)), ((), ())),
                           precision=lax.Precision.HIGHEST,
                           preferred_element_type=_f32)


def _tc1_body(x_ref, w1_ref, degp_ref, ones_ref, ls_ref, dinv_ref):
    deg = _dot_t(degp_ref[...], ones_ref[...])[:N_NODES] + 1.0
    dinv = lax.rsqrt(deg)
    lin = _dot(x_ref[...], w1_ref[...])
    ls_ref[...] = lin * dinv
    dinv_ref[...] = dinv


_tc1_call = pl.pallas_call(
    _tc1_body,
    out_shape=[jax.ShapeDtypeStruct((N_NODES, WROW), _f32),
               jax.ShapeDtypeStruct((N_NODES, 1), _f32)],
)


def _tc2_body(a0_ref, a1_ref, ls1_ref, dinv_ref, b1_ref, w2_ref, ls2_ref):
    dinv = dinv_ref[...]
    a = a0_ref[:N_NODES, :HIDDEN] + a1_ref[:N_NODES, :HIDDEN]
    h = dinv * (a + ls1_ref[:N_NODES, :HIDDEN]) + b1_ref[...]
    h = jnp.maximum(h, 0.0)
    ls2_ref[...] = _dot(h, w2_ref[...]) * dinv


_tc2_call = pl.pallas_call(
    _tc2_body,
    out_shape=jax.ShapeDtypeStruct((N_NODES, WROW), _f32),
)


def _tc3_body(a0_ref, a1_ref, ls2_ref, dinv_ref, b2_ref, batch_ref,
              wfc_ref, bfc_ref, out_ref):
    dinv = dinv_ref[...]
    a = a0_ref[:N_NODES, :HIDDEN] + a1_ref[:N_NODES, :HIDDEN]
    h = dinv * (a + ls2_ref[:N_NODES, :HIDDEN]) + b2_ref[...]
    h = jnp.maximum(h, 0.0)
    gid = lax.broadcasted_iota(_i32, (NUM_GRAPHS, N_NODES), 0)
    mask = (batch_ref[...] == gid).astype(_f32)
    sums = _dot(mask, h)
    cnts = jnp.sum(mask, axis=1, keepdims=True)
    pooled = sums / jnp.maximum(cnts, 1.0)
    out_ref[...] = jnp.maximum(_dot(pooled, wfc_ref[...]) + bfc_ref[...], 0.0)


_tc3_call = pl.pallas_call(
    _tc3_body,
    out_shape=jax.ShapeDtypeStruct((NUM_GRAPHS, OUT_CH), _f32),
)


def kernel(x, edge_index, batch, W1, b1, W2, b2, Wfc, bfc):
    src = edge_index[0]
    dst = edge_index[1]
    src_p = jnp.concatenate([src, jnp.zeros((N_EPAD,), _i32)])
    junk = N_NODES + jnp.arange(N_EPAD, dtype=_i32) % (NPAD - N_NODES)
    dst_p = jnp.concatenate([dst, junk])
    src3 = src_p.reshape(NW, NCHUNK, K)
    dst3 = dst_p.reshape(NW, NCHUNK, K)

    dst2 = dst_p.reshape(NW, EPWP)
    zrows = jnp.zeros((HRPT, WROW), _f32)
    ones_nw = jnp.ones((NW, 1), _f32)

    degp = _deg_call(dst2)
    ls1, dinv = _tc1_call(x, W1, degp, ones_nw)
    acc1 = _agg_call(ls1, src3, dst3, zrows)
    ls2 = _tc2_call(acc1[0], acc1[1], ls1, dinv, b1.reshape(1, -1), W2)
    acc2 = _agg_call(ls2, src3, dst3, zrows)
    out = _tc3_call(acc2[0], acc2[1], ls2, dinv, b2.reshape(1, -1),
                    batch.reshape(1, -1), Wfc, bfc.reshape(1, -1))
    return out


# speedup vs baseline: 1.9324x; 1.9096x over previous
import jax
import jax.numpy as jnp
from jax import lax
from jax.experimental import pallas as pl
from jax.experimental.pallas import tpu as pltpu
from jax.experimental.pallas import tpu_sc as plsc

N_NODES = 10000
N_EDGES = 320000
IN_CH = 128
HIDDEN = 64
OUT_CH = 32
NUM_GRAPHS = 128

NC = 2
NS = 16
NW = NC * NS
EPW = N_EDGES // NW
K = 100
EPWP = EPW
NCHUNK = EPWP // K
NPAD = 10240
RPT = NPAD // NS
HRPT = RPT // 2
WROW = HIDDEN

_f32 = jnp.float32
_i32 = jnp.int32

_sc_mesh = plsc.VectorSubcoreMesh(core_axis_name="c", subcore_axis_name="s")
_sc_params = pltpu.CompilerParams(needs_layout_passes=False,
                                 use_tc_tiling_on_sc=False)


def _deg_body(dst_hbm, out_hbm, dstv, hist):
    c = lax.axis_index("c")
    s = lax.axis_index("s")
    w = c * NS + s

    zero = jnp.zeros((16,), _f32)

    def z(i, carry):
        hist[pl.ds(i * 16, 16)] = zero
        return carry

    lax.fori_loop(0, NPAD // 16, z, 0)
    pltpu.sync_copy(dst_hbm.at[w], dstv)

    ones = jnp.full((16,), 1.0, _f32)

    def body(e, carry):
        idx = dstv[pl.ds(e * 16, 16)]
        plsc.addupdate_scatter(hist, [idx], ones)
        return carry

    lax.fori_loop(0, EPWP // 16, body, 0)
    pltpu.sync_copy(hist, out_hbm.at[w])


_deg_call = pl.kernel(
    _deg_body,
    out_type=jax.ShapeDtypeStruct((NW, NPAD), _f32),
    mesh=_sc_mesh,
    compiler_params=_sc_params,
    scratch_types=[
        pltpu.VMEM((EPWP,), _i32),
        pltpu.VMEM((NPAD,), _f32),
    ],
)


def _agg_body(ls_hbm, src_hbm, dst_hbm, zrows_hbm, out_hbm,
              idxv, dstv, rows0, rows1, obuf, acc, gs0, gs1):
    c = lax.axis_index("c")
    s = lax.axis_index("s")
    w = c * NS + s

    pltpu.sync_copy(zrows_hbm, obuf)
    for m in range(2):
        pltpu.sync_copy(obuf, acc.at[pl.ds(s * RPT + m * HRPT, HRPT)])
    pltpu.sync_copy(src_hbm.at[w], idxv)
    pltpu.sync_copy(dst_hbm.at[w], dstv)
    plsc.subcore_barrier()

    def pair(i, carry):
        j0 = 2 * i
        d0 = pltpu.async_copy(ls_hbm.at[idxv.at[j0]], rows0, gs0)
        d1 = pltpu.async_copy(ls_hbm.at[idxv.at[j0 + 1]], rows1, gs1)
        d0.wait()
        pltpu.sync_copy(rows0, acc.at[dstv.at[j0]], add=True)
        d1.wait()
        pltpu.sync_copy(rows1, acc.at[dstv.at[j0 + 1]], add=True)
        return carry

    lax.fori_loop(0, NCHUNK // 2, pair, 0)
    plsc.subcore_barrier()

    for m in range(2):
        pltpu.sync_copy(acc.at[pl.ds(s * RPT + m * HRPT, HRPT)], obuf)
        pltpu.sync_copy(obuf, out_hbm.at[c, pl.ds(s * RPT + m * HRPT, HRPT)])


_agg_call = pl.kernel(
    _agg_body,
    out_type=jax.ShapeDtypeStruct((NC, NPAD, WROW), _f32),
    mesh=_sc_mesh,
    compiler_params=_sc_params,
    scratch_types=[
        pltpu.VMEM((NCHUNK, K), _i32),
        pltpu.VMEM((NCHUNK, K), _i32),
        pltpu.VMEM((K, WROW), _f32),
        pltpu.VMEM((K, WROW), _f32),
        pltpu.VMEM((HRPT, WROW), _f32),
        pltpu.VMEM_SHARED((NPAD, WROW), _f32),
        pltpu.SemaphoreType.DMA,
        pltpu.SemaphoreType.DMA,
    ],
)


def _dot(a, b):
    return lax.dot_general(a, b, (((1,), (0,)), ((), ())),
                           precision=lax.Precision.HIGHEST,
                           preferred_element_type=_f32)


def _dot_t(a, b):
    return lax.dot_general(a, b, (((0,), (0,)), ((), ())),
                           precision=lax.Precision.HIGHEST,
                           preferred_element_type=_f32)


def _tc1_body(x_ref, w1_ref, degp_ref, ones_ref, ls_ref, dinv_ref):
    deg = _dot_t(degp_ref[...], ones_ref[...])[:N_NODES] + 1.0
    dinv = lax.rsqrt(deg)
    lin = _dot(x_ref[...], w1_ref[...])
    ls_ref[...] = lin * dinv
    dinv_ref[...] = dinv


_tc1_call = pl.pallas_call(
    _tc1_body,
    out_shape=[jax.ShapeDtypeStruct((N_NODES, WROW), _f32),
               jax.ShapeDtypeStruct((N_NODES, 1), _f32)],
)


def _tc2_body(a0_ref, a1_ref, ls1_ref, dinv_ref, b1_ref, w2_ref, ls2_ref):
    dinv = dinv_ref[...]
    a = a0_ref[:N_NODES, :HIDDEN] + a1_ref[:N_NODES, :HIDDEN]
    h = dinv * (a + ls1_ref[:N_NODES, :HIDDEN]) + b1_ref[...]
    h = jnp.maximum(h, 0.0)
    ls2_ref[...] = _dot(h, w2_ref[...]) * dinv


_tc2_call = pl.pallas_call(
    _tc2_body,
    out_shape=jax.ShapeDtypeStruct((N_NODES, WROW), _f32),
)


def _tc3_body(a0_ref, a1_ref, ls2_ref, dinv_ref, b2_ref, batch_ref,
              wfc_ref, bfc_ref, out_ref):
    dinv = dinv_ref[...]
    a = a0_ref[:N_NODES, :HIDDEN] + a1_ref[:N_NODES, :HIDDEN]
    h = dinv * (a + ls2_ref[:N_NODES, :HIDDEN]) + b2_ref[...]
    h = jnp.maximum(h, 0.0)
    gid = lax.broadcasted_iota(_i32, (NUM_GRAPHS, N_NODES), 0)
    mask = (batch_ref[...] == gid).astype(_f32)
    sums = _dot(mask, h)
    cnts = jnp.sum(mask, axis=1, keepdims=True)
    pooled = sums / jnp.maximum(cnts, 1.0)
    out_ref[...] = jnp.maximum(_dot(pooled, wfc_ref[...]) + bfc_ref[...], 0.0)


_tc3_call = pl.pallas_call(
    _tc3_body,
    out_shape=jax.ShapeDtypeStruct((NUM_GRAPHS, OUT_CH), _f32),
)


def kernel(x, edge_index, batch, W1, b1, W2, b2, Wfc, bfc):
    src = edge_index[0]
    dst = edge_index[1]
    src3 = src.reshape(NW, NCHUNK, K)
    dst3 = dst.reshape(NW, NCHUNK, K)

    dst2 = dst.reshape(NW, EPWP)
    zrows = jnp.zeros((HRPT, WROW), _f32)
    ones_nw = jnp.ones((NW, 1), _f32)

    degp = _deg_call(dst2)
    ls1, dinv = _tc1_call(x, W1, degp, ones_nw)
    acc1 = _agg_call(ls1, src3, dst3, zrows)
    ls2 = _tc2_call(acc1[0], acc1[1], ls1, dinv, b1.reshape(1, -1), W2)
    acc2 = _agg_call(ls2, src3, dst3, zrows)
    out = _tc3_call(acc2[0], acc2[1], ls2, dinv, b2.reshape(1, -1),
                    batch.reshape(1, -1), Wfc, bfc.reshape(1, -1))
    return out


# speedup vs baseline: 2.4225x; 1.2536x over previous
import jax
import jax.numpy as jnp
from jax import lax
from jax.experimental import pallas as pl
from jax.experimental.pallas import tpu as pltpu
from jax.experimental.pallas import tpu_sc as plsc

N_NODES = 10000
N_EDGES = 320000
IN_CH = 128
HIDDEN = 64
OUT_CH = 32
NUM_GRAPHS = 128

NC = 2
NS = 16
NW = NC * NS
EPW = N_EDGES // NW
K = 100
EPWP = EPW
NCHUNK = EPWP // K
NBUF = 4
SBYTES = K * HIDDEN * 4
NPAD = 10240
RPT = NPAD // NS
HRPT = RPT // 2
WROW = HIDDEN

_f32 = jnp.float32
_i32 = jnp.int32

_sc_mesh = plsc.VectorSubcoreMesh(core_axis_name="c", subcore_axis_name="s")
_sc_params = pltpu.CompilerParams(needs_layout_passes=False,
                                 use_tc_tiling_on_sc=False)


def _deg_body(dst_hbm, out_hbm, dstv, hist):
    c = lax.axis_index("c")
    s = lax.axis_index("s")
    w = c * NS + s

    zero = jnp.zeros((16,), _f32)

    def z(i, carry):
        hist[pl.ds(i * 16, 16)] = zero
        return carry

    lax.fori_loop(0, NPAD // 16, z, 0)
    pltpu.sync_copy(dst_hbm.at[w], dstv)

    ones = jnp.full((16,), 1.0, _f32)

    def body(e, carry):
        idx = dstv[pl.ds(e * 16, 16)]
        plsc.addupdate_scatter(hist, [idx], ones)
        return carry

    lax.fori_loop(0, EPWP // 16, body, 0)
    pltpu.sync_copy(hist, out_hbm.at[w])


_deg_call = pl.kernel(
    _deg_body,
    out_type=jax.ShapeDtypeStruct((NW, NPAD), _f32),
    mesh=_sc_mesh,
    compiler_params=_sc_params,
    scratch_types=[
        pltpu.VMEM((EPWP,), _i32),
        pltpu.VMEM((NPAD,), _f32),
    ],
)


def _agg_body(ls_hbm, src_hbm, dst_hbm, zrows_hbm, out_hbm,
              idxv, dstv, rows0, rows1, rows2, rows3, obuf, acc,
              gs0, gs1, gs2, gs3, ss0, ss1, ss2, ss3):
    c = lax.axis_index("c")
    s = lax.axis_index("s")
    w = c * NS + s
    rows = [rows0, rows1, rows2, rows3]
    gs = [gs0, gs1, gs2, gs3]
    ss = [ss0, ss1, ss2, ss3]

    pltpu.sync_copy(zrows_hbm, obuf)
    for m in range(2):
        pltpu.sync_copy(obuf, acc.at[pl.ds(s * RPT + m * HRPT, HRPT)])
    pltpu.sync_copy(src_hbm.at[w], idxv)
    pltpu.sync_copy(dst_hbm.at[w], dstv)
    plsc.subcore_barrier()

    def ring(i, carry):
        ds = []
        for b in range(NBUF):
            j = i * NBUF + b

            @pl.when(i > 0)
            def _():
                pltpu.make_async_copy(rows[b], acc.at[dstv.at[j]],
                                      ss[b]).wait()

            ds.append(pltpu.async_copy(ls_hbm.at[idxv.at[j]], rows[b], gs[b]))
        for b in range(NBUF):
            j = i * NBUF + b
            ds[b].wait()
            pltpu.async_copy(rows[b], acc.at[dstv.at[j]], ss[b], add=True)
        return carry

    lax.fori_loop(0, NCHUNK // NBUF, ring, 0)
    for b in range(NBUF):
        j = NCHUNK - NBUF + b
        pltpu.make_async_copy(rows[b], acc.at[dstv.at[j]], ss[b]).wait()
    plsc.subcore_barrier()

    for m in range(2):
        pltpu.sync_copy(acc.at[pl.ds(s * RPT + m * HRPT, HRPT)], obuf)
        pltpu.sync_copy(obuf, out_hbm.at[c, pl.ds(s * RPT + m * HRPT, HRPT)])


_agg_call = pl.kernel(
    _agg_body,
    out_type=jax.ShapeDtypeStruct((NC, NPAD, WROW), _f32),
    mesh=_sc_mesh,
    compiler_params=_sc_params,
    scratch_types=[
        pltpu.VMEM((NCHUNK, K), _i32),
        pltpu.VMEM((NCHUNK, K), _i32),
        pltpu.VMEM((K, WROW), _f32),
        pltpu.VMEM((K, WROW), _f32),
        pltpu.VMEM((K, WROW), _f32),
        pltpu.VMEM((K, WROW), _f32),
        pltpu.VMEM((HRPT, WROW), _f32),
        pltpu.VMEM_SHARED((NPAD, WROW), _f32),
        pltpu.SemaphoreType.DMA,
        pltpu.SemaphoreType.DMA,
        pltpu.SemaphoreType.DMA,
        pltpu.SemaphoreType.DMA,
        pltpu.SemaphoreType.DMA,
        pltpu.SemaphoreType.DMA,
        pltpu.SemaphoreType.DMA,
        pltpu.SemaphoreType.DMA,
    ],
)


def _dot(a, b):
    return lax.dot_general(a, b, (((1,), (0,)), ((), ())),
                           precision=lax.Precision.HIGHEST,
                           preferred_element_type=_f32)


def _dot_t(a, b):
    return lax.dot_general(a, b, (((0,), (0,)), ((), ())),
                           precision=lax.Precision.HIGHEST,
                           preferred_element_type=_f32)


def _tc1_body(x_ref, w1_ref, degp_ref, ones_ref, ls_ref, dinv_ref):
    deg = _dot_t(degp_ref[...], ones_ref[...])[:N_NODES] + 1.0
    dinv = lax.rsqrt(deg)
    lin = _dot(x_ref[...], w1_ref[...])
    ls_ref[...] = lin * dinv
    dinv_ref[...] = dinv


_tc1_call = pl.pallas_call(
    _tc1_body,
    out_shape=[jax.ShapeDtypeStruct((N_NODES, WROW), _f32),
               jax.ShapeDtypeStruct((N_NODES, 1), _f32)],
)


def _tc2_body(a0_ref, a1_ref, ls1_ref, dinv_ref, b1_ref, w2_ref, ls2_ref):
    dinv = dinv_ref[...]
    a = a0_ref[:N_NODES, :HIDDEN] + a1_ref[:N_NODES, :HIDDEN]
    h = dinv * (a + ls1_ref[:N_NODES, :HIDDEN]) + b1_ref[...]
    h = jnp.maximum(h, 0.0)
    ls2_ref[...] = _dot(h, w2_ref[...]) * dinv


_tc2_call = pl.pallas_call(
    _tc2_body,
    out_shape=jax.ShapeDtypeStruct((N_NODES, WROW), _f32),
)


def _tc3_body(a0_ref, a1_ref, ls2_ref, dinv_ref, b2_ref, batch_ref,
              wfc_ref, bfc_ref, out_ref):
    dinv = dinv_ref[...]
    a = a0_ref[:N_NODES, :HIDDEN] + a1_ref[:N_NODES, :HIDDEN]
    h = dinv * (a + ls2_ref[:N_NODES, :HIDDEN]) + b2_ref[...]
    h = jnp.maximum(h, 0.0)
    gid = lax.broadcasted_iota(_i32, (NUM_GRAPHS, N_NODES), 0)
    mask = (batch_ref[...] == gid).astype(_f32)
    sums = _dot(mask, h)
    cnts = jnp.sum(mask, axis=1, keepdims=True)
    pooled = sums / jnp.maximum(cnts, 1.0)
    out_ref[...] = jnp.maximum(_dot(pooled, wfc_ref[...]) + bfc_ref[...], 0.0)


_tc3_call = pl.pallas_call(
    _tc3_body,
    out_shape=jax.ShapeDtypeStruct((NUM_GRAPHS, OUT_CH), _f32),
)


def kernel(x, edge_index, batch, W1, b1, W2, b2, Wfc, bfc):
    src = edge_index[0]
    dst = edge_index[1]
    src3 = src.reshape(NW, NCHUNK, K)
    dst3 = dst.reshape(NW, NCHUNK, K)

    dst2 = dst.reshape(NW, EPWP)
    zrows = jnp.zeros((HRPT, WROW), _f32)
    ones_nw = jnp.ones((NW, 1), _f32)

    degp = _deg_call(dst2)
    ls1, dinv = _tc1_call(x, W1, degp, ones_nw)
    acc1 = _agg_call(ls1, src3, dst3, zrows)
    ls2 = _tc2_call(acc1[0], acc1[1], ls1, dinv, b1.reshape(1, -1), W2)
    acc2 = _agg_call(ls2, src3, dst3, zrows)
    out = _tc3_call(acc2[0], acc2[1], ls2, dinv, b2.reshape(1, -1),
                    batch.reshape(1, -1), Wfc, bfc.reshape(1, -1))
    return out


# speedup vs baseline: 2.4507x; 1.0116x over previous
import jax
import jax.numpy as jnp
from jax import lax
from jax.experimental import pallas as pl
from jax.experimental.pallas import tpu as pltpu
from jax.experimental.pallas import tpu_sc as plsc

N_NODES = 10000
N_EDGES = 320000
IN_CH = 128
HIDDEN = 64
OUT_CH = 32
NUM_GRAPHS = 128

NC = 2
NS = 16
NW = NC * NS
EPW = N_EDGES // NW
K = 100
EPWP = EPW
NCHUNK = EPWP // K
NBUF = 5
SBYTES = K * HIDDEN * 4
NPAD = 10240
RPT = NPAD // NS
HRPT = RPT // 2
WROW = HIDDEN

_f32 = jnp.float32
_i32 = jnp.int32

_sc_mesh = plsc.VectorSubcoreMesh(core_axis_name="c", subcore_axis_name="s")
_sc_params = pltpu.CompilerParams(needs_layout_passes=False,
                                 use_tc_tiling_on_sc=False)


def _deg_body(dst_hbm, out_hbm, dstv, hist):
    c = lax.axis_index("c")
    s = lax.axis_index("s")
    w = c * NS + s

    zero = jnp.zeros((16,), _f32)

    def z(i, carry):
        hist[pl.ds(i * 16, 16)] = zero
        return carry

    lax.fori_loop(0, NPAD // 16, z, 0)
    pltpu.sync_copy(dst_hbm.at[w], dstv)

    ones = jnp.full((16,), 1.0, _f32)

    def body(e, carry):
        idx = dstv[pl.ds(e * 16, 16)]
        plsc.addupdate_scatter(hist, [idx], ones)
        return carry

    lax.fori_loop(0, EPWP // 16, body, 0)
    pltpu.sync_copy(hist, out_hbm.at[w])


_deg_call = pl.kernel(
    _deg_body,
    out_type=jax.ShapeDtypeStruct((NW, NPAD), _f32),
    mesh=_sc_mesh,
    compiler_params=_sc_params,
    scratch_types=[
        pltpu.VMEM((EPWP,), _i32),
        pltpu.VMEM((NPAD,), _f32),
    ],
)


def _agg_body(ls_hbm, src_hbm, dst_hbm, zrows_hbm, out_hbm,
              idxv, dstv, rows0, rows1, rows2, rows3, rows4, obuf, acc,
              gs0, gs1, gs2, gs3, gs4, ss0, ss1, ss2, ss3, ss4):
    c = lax.axis_index("c")
    s = lax.axis_index("s")
    w = c * NS + s
    rows = [rows0, rows1, rows2, rows3, rows4]
    gs = [gs0, gs1, gs2, gs3, gs4]
    ss = [ss0, ss1, ss2, ss3, ss4]

    pltpu.sync_copy(zrows_hbm, obuf)
    for m in range(2):
        pltpu.sync_copy(obuf, acc.at[pl.ds(s * RPT + m * HRPT, HRPT)])
    pltpu.sync_copy(src_hbm.at[w], idxv)
    pltpu.sync_copy(dst_hbm.at[w], dstv)
    plsc.subcore_barrier()

    def ring(i, carry):
        ds = []
        for b in range(NBUF):
            j = i * NBUF + b

            @pl.when(i > 0)
            def _():
                pltpu.make_async_copy(rows[b], acc.at[dstv.at[j]],
                                      ss[b]).wait()

            ds.append(pltpu.async_copy(ls_hbm.at[idxv.at[j]], rows[b], gs[b]))
        for b in range(NBUF):
            j = i * NBUF + b
            ds[b].wait()
            pltpu.async_copy(rows[b], acc.at[dstv.at[j]], ss[b], add=True)
        return carry

    lax.fori_loop(0, NCHUNK // NBUF, ring, 0)
    for b in range(NBUF):
        j = NCHUNK - NBUF + b
        pltpu.make_async_copy(rows[b], acc.at[dstv.at[j]], ss[b]).wait()
    plsc.subcore_barrier()

    for m in range(2):
        pltpu.sync_copy(acc.at[pl.ds(s * RPT + m * HRPT, HRPT)], obuf)
        pltpu.sync_copy(obuf, out_hbm.at[c, pl.ds(s * RPT + m * HRPT, HRPT)])


_agg_call = pl.kernel(
    _agg_body,
    out_type=jax.ShapeDtypeStruct((NC, NPAD, WROW), _f32),
    mesh=_sc_mesh,
    compiler_params=_sc_params,
    scratch_types=[
        pltpu.VMEM((NCHUNK, K), _i32),
        pltpu.VMEM((NCHUNK, K), _i32),
        pltpu.VMEM((K, WROW), _f32),
        pltpu.VMEM((K, WROW), _f32),
        pltpu.VMEM((K, WROW), _f32),
        pltpu.VMEM((K, WROW), _f32),
        pltpu.VMEM((K, WROW), _f32),
        pltpu.VMEM((HRPT, WROW), _f32),
        pltpu.VMEM_SHARED((NPAD, WROW), _f32),
        pltpu.SemaphoreType.DMA,
        pltpu.SemaphoreType.DMA,
        pltpu.SemaphoreType.DMA,
        pltpu.SemaphoreType.DMA,
        pltpu.SemaphoreType.DMA,
        pltpu.SemaphoreType.DMA,
        pltpu.SemaphoreType.DMA,
        pltpu.SemaphoreType.DMA,
        pltpu.SemaphoreType.DMA,
        pltpu.SemaphoreType.DMA,
    ],
)


def _dot(a, b):
    return lax.dot_general(a, b, (((1,), (0,)), ((), ())),
                           precision=lax.Precision.HIGHEST,
                           preferred_element_type=_f32)


def _dot_t(a, b):
    return lax.dot_general(a, b, (((0,), (0,)), ((), ())),
                           precision=lax.Precision.HIGHEST,
                           preferred_element_type=_f32)


def _tc1_body(x_ref, w1_ref, degp_ref, ones_ref, ls_ref, dinv_ref):
    deg = _dot_t(degp_ref[...], ones_ref[...])[:N_NODES] + 1.0
    dinv = lax.rsqrt(deg)
    lin = _dot(x_ref[...], w1_ref[...])
    ls_ref[...] = lin * dinv
    dinv_ref[...] = dinv


_tc1_call = pl.pallas_call(
    _tc1_body,
    out_shape=[jax.ShapeDtypeStruct((N_NODES, WROW), _f32),
               jax.ShapeDtypeStruct((N_NODES, 1), _f32)],
)


def _tc2_body(a0_ref, a1_ref, ls1_ref, dinv_ref, b1_ref, w2_ref, ls2_ref):
    dinv = dinv_ref[...]
    a = a0_ref[:N_NODES, :HIDDEN] + a1_ref[:N_NODES, :HIDDEN]
    h = dinv * (a + ls1_ref[:N_NODES, :HIDDEN]) + b1_ref[...]
    h = jnp.maximum(h, 0.0)
    ls2_ref[...] = _dot(h, w2_ref[...]) * dinv


_tc2_call = pl.pallas_call(
    _tc2_body,
    out_shape=jax.ShapeDtypeStruct((N_NODES, WROW), _f32),
)


def _tc3_body(a0_ref, a1_ref, ls2_ref, dinv_ref, b2_ref, batch_ref,
              wfc_ref, bfc_ref, out_ref):
    dinv = dinv_ref[...]
    a = a0_ref[:N_NODES, :HIDDEN] + a1_ref[:N_NODES, :HIDDEN]
    h = dinv * (a + ls2_ref[:N_NODES, :HIDDEN]) + b2_ref[...]
    h = jnp.maximum(h, 0.0)
    gid = lax.broadcasted_iota(_i32, (NUM_GRAPHS, N_NODES), 0)
    mask = (batch_ref[...] == gid).astype(_f32)
    sums = _dot(mask, h)
    cnts = jnp.sum(mask, axis=1, keepdims=True)
    pooled = sums / jnp.maximum(cnts, 1.0)
    out_ref[...] = jnp.maximum(_dot(pooled, wfc_ref[...]) + bfc_ref[...], 0.0)


_tc3_call = pl.pallas_call(
    _tc3_body,
    out_shape=jax.ShapeDtypeStruct((NUM_GRAPHS, OUT_CH), _f32),
)


def kernel(x, edge_index, batch, W1, b1, W2, b2, Wfc, bfc):
    src = edge_index[0]
    dst = edge_index[1]
    src3 = src.reshape(NW, NCHUNK, K)
    dst3 = dst.reshape(NW, NCHUNK, K)

    dst2 = dst.reshape(NW, EPWP)
    zrows = jnp.zeros((HRPT, WROW), _f32)
    ones_nw = jnp.ones((NW, 1), _f32)

    degp = _deg_call(dst2)
    ls1, dinv = _tc1_call(x, W1, degp, ones_nw)
    acc1 = _agg_call(ls1, src3, dst3, zrows)
    ls2 = _tc2_call(acc1[0], acc1[1], ls1, dinv, b1.reshape(1, -1), W2)
    acc2 = _agg_call(ls2, src3, dst3, zrows)
    out = _tc3_call(acc2[0], acc2[1], ls2, dinv, b2.reshape(1, -1),
                    batch.reshape(1, -1), Wfc, bfc.reshape(1, -1))
    return out


# speedup vs baseline: 2.4521x; 1.0006x over previous
import jax
import jax.numpy as jnp
from jax import lax
from jax.experimental import pallas as pl
from jax.experimental.pallas import tpu as pltpu
from jax.experimental.pallas import tpu_sc as plsc

N_NODES = 10000
N_EDGES = 320000
IN_CH = 128
HIDDEN = 64
OUT_CH = 32
NUM_GRAPHS = 128

NC = 2
NS = 16
NW = NC * NS
EPW = N_EDGES // NW
K = 100
EPWP = EPW
NCHUNK = EPWP // K
NBUF = 5
SBYTES = K * HIDDEN * 4
NPAD = 10240
RPT = NPAD // NS
HRPT = RPT // 2
WROW = HIDDEN

_f32 = jnp.float32
_i32 = jnp.int32

_sc_mesh = plsc.VectorSubcoreMesh(core_axis_name="c", subcore_axis_name="s")
_sc_params = pltpu.CompilerParams(needs_layout_passes=False,
                                 use_tc_tiling_on_sc=False,
                                 skip_device_barrier=True)


def _deg_body(dst_hbm, out_hbm, dstv, hist):
    c = lax.axis_index("c")
    s = lax.axis_index("s")
    w = c * NS + s

    zero = jnp.zeros((16,), _f32)

    def z(i, carry):
        hist[pl.ds(i * 16, 16)] = zero
        return carry

    lax.fori_loop(0, NPAD // 16, z, 0)
    pltpu.sync_copy(dst_hbm.at[w], dstv)

    ones = jnp.full((16,), 1.0, _f32)

    def body(e, carry):
        idx = dstv[pl.ds(e * 16, 16)]
        plsc.addupdate_scatter(hist, [idx], ones)
        return carry

    lax.fori_loop(0, EPWP // 16, body, 0)
    pltpu.sync_copy(hist, out_hbm.at[w])


_deg_call = pl.kernel(
    _deg_body,
    out_type=jax.ShapeDtypeStruct((NW, NPAD), _f32),
    mesh=_sc_mesh,
    compiler_params=_sc_params,
    scratch_types=[
        pltpu.VMEM((EPWP,), _i32),
        pltpu.VMEM((NPAD,), _f32),
    ],
)


def _agg_body(ls_hbm, src_hbm, dst_hbm, zrows_hbm, out_hbm,
              idxv, dstv, rows0, rows1, rows2, rows3, rows4, obuf, acc,
              gs0, gs1, gs2, gs3, gs4, ss0, ss1, ss2, ss3, ss4):
    c = lax.axis_index("c")
    s = lax.axis_index("s")
    w = c * NS + s
    rows = [rows0, rows1, rows2, rows3, rows4]
    gs = [gs0, gs1, gs2, gs3, gs4]
    ss = [ss0, ss1, ss2, ss3, ss4]

    pltpu.sync_copy(zrows_hbm, obuf)
    for m in range(2):
        pltpu.sync_copy(obuf, acc.at[pl.ds(s * RPT + m * HRPT, HRPT)])
    pltpu.sync_copy(src_hbm.at[w], idxv)
    pltpu.sync_copy(dst_hbm.at[w], dstv)
    plsc.subcore_barrier()

    def ring(i, carry):
        ds = []
        for b in range(NBUF):
            j = i * NBUF + b

            @pl.when(i > 0)
            def _():
                pltpu.make_async_copy(rows[b], acc.at[dstv.at[j]],
                                      ss[b]).wait()

            ds.append(pltpu.async_copy(ls_hbm.at[idxv.at[j]], rows[b], gs[b]))
        for b in range(NBUF):
            j = i * NBUF + b
            ds[b].wait()
            pltpu.async_copy(rows[b], acc.at[dstv.at[j]], ss[b], add=True)
        return carry

    lax.fori_loop(0, NCHUNK // NBUF, ring, 0)
    for b in range(NBUF):
        j = NCHUNK - NBUF + b
        pltpu.make_async_copy(rows[b], acc.at[dstv.at[j]], ss[b]).wait()
    plsc.subcore_barrier()

    for m in range(2):
        pltpu.sync_copy(acc.at[pl.ds(s * RPT + m * HRPT, HRPT)], obuf)
        pltpu.sync_copy(obuf, out_hbm.at[c, pl.ds(s * RPT + m * HRPT, HRPT)])


_agg_call = pl.kernel(
    _agg_body,
    out_type=jax.ShapeDtypeStruct((NC, NPAD, WROW), _f32),
    mesh=_sc_mesh,
    compiler_params=_sc_params,
    scratch_types=[
        pltpu.VMEM((NCHUNK, K), _i32),
        pltpu.VMEM((NCHUNK, K), _i32),
        pltpu.VMEM((K, WROW), _f32),
        pltpu.VMEM((K, WROW), _f32),
        pltpu.VMEM((K, WROW), _f32),
        pltpu.VMEM((K, WROW), _f32),
        pltpu.VMEM((K, WROW), _f32),
        pltpu.VMEM((HRPT, WROW), _f32),
        pltpu.VMEM_SHARED((NPAD, WROW), _f32),
        pltpu.SemaphoreType.DMA,
        pltpu.SemaphoreType.DMA,
        pltpu.SemaphoreType.DMA,
        pltpu.SemaphoreType.DMA,
        pltpu.SemaphoreType.DMA,
        pltpu.SemaphoreType.DMA,
        pltpu.SemaphoreType.DMA,
        pltpu.SemaphoreType.DMA,
        pltpu.SemaphoreType.DMA,
        pltpu.SemaphoreType.DMA,
    ],
)


def _dot(a, b):
    return lax.dot_general(a, b, (((1,), (0,)), ((), ())),
                           precision=lax.Precision.HIGHEST,
                           preferred_element_type=_f32)


def _dot_t(a, b):
    return lax.dot_general(a, b, (((0,), (0,)), ((), ())),
                           precision=lax.Precision.HIGHEST,
                           preferred_element_type=_f32)


def _tc1_body(x_ref, w1_ref, degp_ref, ones_ref, ls_ref, dinv_ref):
    deg = _dot_t(degp_ref[...], ones_ref[...])[:N_NODES] + 1.0
    dinv = lax.rsqrt(deg)
    lin = _dot(x_ref[...], w1_ref[...])
    ls_ref[...] = lin * dinv
    dinv_ref[...] = dinv


_tc1_call = pl.pallas_call(
    _tc1_body,
    out_shape=[jax.ShapeDtypeStruct((N_NODES, WROW), _f32),
               jax.ShapeDtypeStruct((N_NODES, 1), _f32)],
)


def _tc2_body(a0_ref, a1_ref, ls1_ref, dinv_ref, b1_ref, w2_ref, ls2_ref):
    dinv = dinv_ref[...]
    a = a0_ref[:N_NODES, :HIDDEN] + a1_ref[:N_NODES, :HIDDEN]
    h = dinv * (a + ls1_ref[:N_NODES, :HIDDEN]) + b1_ref[...]
    h = jnp.maximum(h, 0.0)
    ls2_ref[...] = _dot(h, w2_ref[...]) * dinv


_tc2_call = pl.pallas_call(
    _tc2_body,
    out_shape=jax.ShapeDtypeStruct((N_NODES, WROW), _f32),
)


def _tc3_body(a0_ref, a1_ref, ls2_ref, dinv_ref, b2_ref, batch_ref,
              wfc_ref, bfc_ref, out_ref):
    dinv = dinv_ref[...]
    a = a0_ref[:N_NODES, :HIDDEN] + a1_ref[:N_NODES, :HIDDEN]
    h = dinv * (a + ls2_ref[:N_NODES, :HIDDEN]) + b2_ref[...]
    h = jnp.maximum(h, 0.0)
    gid = lax.broadcasted_iota(_i32, (NUM_GRAPHS, N_NODES), 0)
    mask = (batch_ref[...] == gid).astype(_f32)
    sums = _dot(mask, h)
    cnts = jnp.sum(mask, axis=1, keepdims=True)
    pooled = sums / jnp.maximum(cnts, 1.0)
    out_ref[...] = jnp.maximum(_dot(pooled, wfc_ref[...]) + bfc_ref[...], 0.0)


_tc3_call = pl.pallas_call(
    _tc3_body,
    out_shape=jax.ShapeDtypeStruct((NUM_GRAPHS, OUT_CH), _f32),
)


def kernel(x, edge_index, batch, W1, b1, W2, b2, Wfc, bfc):
    src = edge_index[0]
    dst = edge_index[1]
    src3 = src.reshape(NW, NCHUNK, K)
    dst3 = dst.reshape(NW, NCHUNK, K)

    dst2 = dst.reshape(NW, EPWP)
    zrows = jnp.zeros((HRPT, WROW), _f32)
    ones_nw = jnp.ones((NW, 1), _f32)

    degp = _deg_call(dst2)
    ls1, dinv = _tc1_call(x, W1, degp, ones_nw)
    acc1 = _agg_call(ls1, src3, dst3, zrows)
    ls2 = _tc2_call(acc1[0], acc1[1], ls1, dinv, b1.reshape(1, -1), W2)
    acc2 = _agg_call(ls2, src3, dst3, zrows)
    out = _tc3_call(acc2[0], acc2[1], ls2, dinv, b2.reshape(1, -1),
                    batch.reshape(1, -1), Wfc, bfc.reshape(1, -1))
    return out


# speedup vs baseline: 2.5984x; 1.0597x over previous
import jax
import jax.numpy as jnp
from jax import lax
from jax.experimental import pallas as pl
from jax.experimental.pallas import tpu as pltpu
from jax.experimental.pallas import tpu_sc as plsc

N_NODES = 10000
N_EDGES = 320000
IN_CH = 128
HIDDEN = 64
OUT_CH = 32
NUM_GRAPHS = 128

NC = 2
NS = 16
NW = NC * NS
EPW = N_EDGES // NW
K = 100
EPWP = EPW
NCHUNK = EPWP // K
NBUF = 5
SBYTES = K * HIDDEN * 4
NPAD = 10240
RPT = NPAD // NS
HRPT = RPT // 2
WROW = HIDDEN

_f32 = jnp.float32
_i32 = jnp.int32

_sc_mesh = plsc.VectorSubcoreMesh(core_axis_name="c", subcore_axis_name="s")
_sc_params = pltpu.CompilerParams(needs_layout_passes=False,
                                 use_tc_tiling_on_sc=False)


def _deg_body(dst_hbm, out_hbm, dstv, hist):
    c = lax.axis_index("c")
    s = lax.axis_index("s")
    w = c * NS + s

    zero = jnp.zeros((16,), _f32)

    def z(i, carry):
        hist[pl.ds(i * 16, 16)] = zero
        return carry

    lax.fori_loop(0, NPAD // 16, z, 0)
    pltpu.sync_copy(dst_hbm.at[w], dstv)

    ones = jnp.full((16,), 1.0, _f32)

    def body(e, carry):
        idx = dstv[pl.ds(e * 16, 16)]
        plsc.addupdate_scatter(hist, [idx], ones)
        return carry

    lax.fori_loop(0, EPWP // 16, body, 0)
    pltpu.sync_copy(hist, out_hbm.at[w])


_deg_call = pl.kernel(
    _deg_body,
    out_type=jax.ShapeDtypeStruct((NW, NPAD), _f32),
    mesh=_sc_mesh,
    compiler_params=_sc_params,
    scratch_types=[
        pltpu.VMEM((EPWP,), _i32),
        pltpu.VMEM((NPAD,), _f32),
    ],
)


def _agg_body(ls_hbm, src_hbm, dst_hbm, zrows_hbm, out_hbm,
              idxv, dstv, rows0, rows1, rows2, rows3, rows4, obuf, acc,
              gs0, gs1, gs2, gs3, gs4, ss0, ss1, ss2, ss3, ss4):
    c = lax.axis_index("c")
    s = lax.axis_index("s")
    w = c * NS + s
    rows = [rows0, rows1, rows2, rows3, rows4]
    gs = [gs0, gs1, gs2, gs3, gs4]
    ss = [ss0, ss1, ss2, ss3, ss4]

    pltpu.sync_copy(zrows_hbm, obuf)
    for m in range(2):
        pltpu.sync_copy(obuf, acc.at[pl.ds(s * RPT + m * HRPT, HRPT)])
    pltpu.sync_copy(src_hbm.at[w], idxv)
    pltpu.sync_copy(dst_hbm.at[w], dstv)
    plsc.subcore_barrier()

    def ring(i, carry):
        ds = []
        for b in range(NBUF):
            j = i * NBUF + b

            @pl.when(i > 0)
            def _():
                pltpu.make_async_copy(rows[b], acc.at[dstv.at[j]],
                                      ss[b]).wait()

            ds.append(pltpu.async_copy(ls_hbm.at[idxv.at[j]], rows[b], gs[b]))
        for b in range(NBUF):
            j = i * NBUF + b
            ds[b].wait()
            pltpu.async_copy(rows[b], acc.at[dstv.at[j]], ss[b], add=True)
        return carry

    lax.fori_loop(0, NCHUNK // NBUF, ring, 0)
    for b in range(NBUF):
        j = NCHUNK - NBUF + b
        pltpu.make_async_copy(rows[b], acc.at[dstv.at[j]], ss[b]).wait()
    plsc.subcore_barrier()

    for m in range(2):
        pltpu.sync_copy(acc.at[pl.ds(s * RPT + m * HRPT, HRPT)], obuf)
        pltpu.sync_copy(obuf, out_hbm.at[c, pl.ds(s * RPT + m * HRPT, HRPT)])


_agg_call = pl.kernel(
    _agg_body,
    out_type=jax.ShapeDtypeStruct((NC, NPAD, WROW), _f32),
    mesh=_sc_mesh,
    compiler_params=_sc_params,
    scratch_types=[
        pltpu.VMEM((NCHUNK, K), _i32),
        pltpu.VMEM((NCHUNK, K), _i32),
        pltpu.VMEM((K, WROW), _f32),
        pltpu.VMEM((K, WROW), _f32),
        pltpu.VMEM((K, WROW), _f32),
        pltpu.VMEM((K, WROW), _f32),
        pltpu.VMEM((K, WROW), _f32),
        pltpu.VMEM((HRPT, WROW), _f32),
        pltpu.VMEM_SHARED((NPAD, WROW), _f32),
        pltpu.SemaphoreType.DMA,
        pltpu.SemaphoreType.DMA,
        pltpu.SemaphoreType.DMA,
        pltpu.SemaphoreType.DMA,
        pltpu.SemaphoreType.DMA,
        pltpu.SemaphoreType.DMA,
        pltpu.SemaphoreType.DMA,
        pltpu.SemaphoreType.DMA,
        pltpu.SemaphoreType.DMA,
        pltpu.SemaphoreType.DMA,
    ],
)


def _dot(a, b):
    return lax.dot_general(a, b, (((1,), (0,)), ((), ())),
                           precision=lax.Precision.HIGHEST,
                           preferred_element_type=_f32)


def _dot_t(a, b):
    return lax.dot_general(a, b, (((0,), (0,)), ((), ())),
                           precision=lax.Precision.HIGHEST,
                           preferred_element_type=_f32)


def _tc1_body(x_ref, w1_ref, degp_ref, ones_ref, ls_ref, dinv_ref):
    deg = _dot_t(degp_ref[...], ones_ref[...])[:N_NODES] + 1.0
    dinv = lax.rsqrt(deg)
    lin = _dot(x_ref[...], w1_ref[...])
    ls_ref[...] = lin * dinv
    dinv_ref[...] = dinv


_tc1_call = pl.pallas_call(
    _tc1_body,
    out_shape=[jax.ShapeDtypeStruct((N_NODES, WROW), _f32),
               jax.ShapeDtypeStruct((N_NODES, 1), _f32)],
)


def _tc2_body(acc_ref, ls1_ref, dinv_ref, b1_ref, w2_ref, ls2_ref):
    dinv = dinv_ref[...]
    a = acc_ref[0, :N_NODES, :HIDDEN] + acc_ref[1, :N_NODES, :HIDDEN]
    h = dinv * (a + ls1_ref[:N_NODES, :HIDDEN]) + b1_ref[...]
    h = jnp.maximum(h, 0.0)
    ls2_ref[...] = _dot(h, w2_ref[...]) * dinv


_tc2_call = pl.pallas_call(
    _tc2_body,
    out_shape=jax.ShapeDtypeStruct((N_NODES, WROW), _f32),
)


def _tc3_body(acc_ref, ls2_ref, dinv_ref, b2_ref, batch_ref,
              wfc_ref, bfc_ref, out_ref):
    dinv = dinv_ref[...]
    a = acc_ref[0, :N_NODES, :HIDDEN] + acc_ref[1, :N_NODES, :HIDDEN]
    h = dinv * (a + ls2_ref[:N_NODES, :HIDDEN]) + b2_ref[...]
    h = jnp.maximum(h, 0.0)
    gid = lax.broadcasted_iota(_i32, (NUM_GRAPHS, N_NODES), 0)
    mask = (batch_ref[...] == gid).astype(_f32)
    sums = _dot(mask, h)
    cnts = jnp.sum(mask, axis=1, keepdims=True)
    pooled = sums / jnp.maximum(cnts, 1.0)
    out_ref[...] = jnp.maximum(_dot(pooled, wfc_ref[...]) + bfc_ref[...], 0.0)


_tc3_call = pl.pallas_call(
    _tc3_body,
    out_shape=jax.ShapeDtypeStruct((NUM_GRAPHS, OUT_CH), _f32),
)


def kernel(x, edge_index, batch, W1, b1, W2, b2, Wfc, bfc):
    src = edge_index[0]
    dst = edge_index[1]
    src3 = src.reshape(NW, NCHUNK, K)
    dst3 = dst.reshape(NW, NCHUNK, K)

    dst2 = dst.reshape(NW, EPWP)
    zrows = jnp.zeros((HRPT, WROW), _f32)
    ones_nw = jnp.ones((NW, 1), _f32)

    degp = _deg_call(dst2)
    ls1, dinv = _tc1_call(x, W1, degp, ones_nw)
    acc1 = _agg_call(ls1, src3, dst3, zrows)
    ls2 = _tc2_call(acc1, ls1, dinv, b1.reshape(1, -1), W2)
    acc2 = _agg_call(ls2, src3, dst3, zrows)
    out = _tc3_call(acc2, ls2, dinv, b2.reshape(1, -1),
                    batch.reshape(1, -1), Wfc, bfc.reshape(1, -1))
    return out
